# baseline (speedup 1.0000x reference)
import functools

import jax
import jax.numpy as jnp
import numpy as np
from jax import lax
from jax.experimental import pallas as pl
from jax.experimental.pallas import tpu as pltpu
from jax.experimental.pallas import tpu_sc as plsc

DIM = 64
MAX_POS = 8192
THETA = 10000.0
SEQ = 8192

NC = 2
NS = 16
NW = NC * NS
BPW = SEQ // NW
CHUNK = 128
NCH = BPW // CHUNK


def _fused_table():
    inv_freq = 1.0 / (THETA ** (np.arange(0, DIM, 2, dtype=np.float32) / DIM))
    t = np.arange(MAX_POS, dtype=np.float32)
    freqs = (t[:, None] * inv_freq[None, :]).astype(np.float32)
    emb = np.concatenate((freqs, freqs), axis=-1)
    return np.concatenate(
        (np.cos(emb), np.sin(emb)), axis=-1).astype(np.float32)


_TAB = _fused_table()

_mesh = plsc.VectorSubcoreMesh(core_axis_name="c", subcore_axis_name="s")


@functools.partial(
    pl.kernel,
    mesh=_mesh,
    out_type=jax.ShapeDtypeStruct((SEQ, 2 * DIM), jnp.float32),
    scratch_types=[
        pltpu.VMEM((BPW,), jnp.int32),
        pltpu.VMEM((BPW, 2 * DIM), jnp.float32),
        pltpu.SemaphoreType.DMA,
    ],
)
def _rope_gather(tab_hbm, idx_hbm, wide_out, idx_v, rows_v, gsem):
    wid = lax.axis_index("s") * NC + lax.axis_index("c")
    base = wid * BPW
    pltpu.sync_copy(idx_hbm.at[0, pl.ds(base, BPW)], idx_v)
    copies = []
    for j in range(NCH):
        idx_sl = idx_v.at[pl.ds(j * CHUNK, CHUNK)]
        copies.append(pltpu.async_copy(
            tab_hbm.at[idx_sl], rows_v.at[pl.ds(j * CHUNK, CHUNK)], gsem))
    for c in copies:
        c.wait()
    pltpu.sync_copy(rows_v, wide_out.at[pl.ds(base, BPW)])


_SPLIT_ROWS = 1024


def _split_body(wide_ref, cos_ref, sin_ref):
    w = wide_ref[...]
    cos_ref[...] = w[:, :DIM]
    sin_ref[...] = w[:, DIM:]


_split = pl.pallas_call(
    _split_body,
    grid=(SEQ // _SPLIT_ROWS,),
    in_specs=[pl.BlockSpec((_SPLIT_ROWS, 2 * DIM), lambda i: (i, 0))],
    out_specs=[
        pl.BlockSpec((_SPLIT_ROWS, DIM), lambda i: (i, 0)),
        pl.BlockSpec((_SPLIT_ROWS, DIM), lambda i: (i, 0)),
    ],
    out_shape=[
        jax.ShapeDtypeStruct((SEQ, DIM), jnp.float32),
        jax.ShapeDtypeStruct((SEQ, DIM), jnp.float32),
    ],
)


def kernel(x, position_ids):
    tab = jnp.asarray(_TAB)
    idx = position_ids.astype(jnp.int32)
    wide = _rope_gather(tab, idx)
    cos, sin = _split(wide)
    cos = cos.reshape(1, 1, SEQ, DIM).astype(x.dtype)
    sin = sin.reshape(1, 1, SEQ, DIM).astype(x.dtype)
    return (cos, sin)


# speedup vs baseline: 1.1805x; 1.1805x over previous
import functools

import jax
import jax.numpy as jnp
import numpy as np
from jax import lax
from jax.experimental import pallas as pl
from jax.experimental.pallas import tpu as pltpu
from jax.experimental.pallas import tpu_sc as plsc

DIM = 64
MAX_POS = 8192
THETA = 10000.0
SEQ = 8192

NC = 2
NS = 16
NW = NC * NS
BPW = SEQ // NW
CHUNK = 128
NCH = BPW // CHUNK


def _fused_table():
    inv_freq = 1.0 / (THETA ** (np.arange(0, DIM, 2, dtype=np.float32) / DIM))
    t = np.arange(MAX_POS, dtype=np.float32)
    freqs = (t[:, None] * inv_freq[None, :]).astype(np.float32)
    emb = np.concatenate((freqs, freqs), axis=-1)
    return np.concatenate(
        (np.cos(emb), np.sin(emb)), axis=-1).astype(np.float32)


_TAB = _fused_table()

_mesh = plsc.VectorSubcoreMesh(core_axis_name="c", subcore_axis_name="s")


@functools.partial(
    pl.kernel,
    mesh=_mesh,
    out_type=jax.ShapeDtypeStruct((SEQ, 2 * DIM), jnp.float32),
    scratch_types=[
        pltpu.VMEM((BPW,), jnp.int32),
        pltpu.VMEM((BPW, 2 * DIM), jnp.float32),
        pltpu.SemaphoreType.DMA,
    ],
)
def _rope_gather(tab_hbm, idx_hbm, wide_out, idx_v, rows_v, gsem):
    wid = lax.axis_index("s") * NC + lax.axis_index("c")
    base = wid * BPW
    pltpu.sync_copy(idx_hbm.at[0, pl.ds(base, BPW)], idx_v)
    copies = []
    for j in range(NCH):
        idx_sl = idx_v.at[pl.ds(j * CHUNK, CHUNK)]
        copies.append(pltpu.async_copy(
            tab_hbm.at[idx_sl], rows_v.at[pl.ds(j * CHUNK, CHUNK)], gsem))
    for c in copies:
        c.wait()
    pltpu.sync_copy(rows_v, wide_out.at[pl.ds(base, BPW)])


_TR_ROWS = 1024


def _split_t_body(wide_ref, cos_ref, sin_ref):
    w = wide_ref[...]
    cos_ref[...] = w[:, :DIM].T
    sin_ref[...] = w[:, DIM:].T


_split_t = pl.pallas_call(
    _split_t_body,
    grid=(SEQ // _TR_ROWS,),
    in_specs=[pl.BlockSpec((_TR_ROWS, 2 * DIM), lambda i: (i, 0))],
    out_specs=[
        pl.BlockSpec((DIM, _TR_ROWS), lambda i: (0, i)),
        pl.BlockSpec((DIM, _TR_ROWS), lambda i: (0, i)),
    ],
    out_shape=[
        jax.ShapeDtypeStruct((DIM, SEQ), jnp.float32),
        jax.ShapeDtypeStruct((DIM, SEQ), jnp.float32),
    ],
)


def kernel(x, position_ids):
    tab = jnp.asarray(_TAB)
    idx = position_ids.astype(jnp.int32)
    wide = _rope_gather(tab, idx)
    cos_t, sin_t = _split_t(wide)
    cos = cos_t.T.reshape(1, 1, SEQ, DIM).astype(x.dtype)
    sin = sin_t.T.reshape(1, 1, SEQ, DIM).astype(x.dtype)
    return (cos, sin)


# speedup vs baseline: 1.2105x; 1.0255x over previous
import functools

import jax
import jax.numpy as jnp
import numpy as np
from jax import lax
from jax.experimental import pallas as pl
from jax.experimental.pallas import tpu as pltpu
from jax.experimental.pallas import tpu_sc as plsc

DIM = 64
MAX_POS = 8192
THETA = 10000.0
SEQ = 8192

NC = 2
NS = 16
NW = NC * NS
BPW = SEQ // NW
CHUNK = 128
NCH = BPW // CHUNK


def _fused_table():
    inv_freq = 1.0 / (THETA ** (np.arange(0, DIM, 2, dtype=np.float32) / DIM))
    t = np.arange(MAX_POS, dtype=np.float32)
    freqs = (t[:, None] * inv_freq[None, :]).astype(np.float32)
    emb = np.concatenate((freqs, freqs), axis=-1)
    tab = np.concatenate(
        (np.cos(emb), np.sin(emb)), axis=-1).astype(np.float32)
    return np.concatenate(
        (tab, np.zeros((8, 2 * DIM), np.float32)), axis=0)


_TAB = _fused_table()

_mesh = plsc.VectorSubcoreMesh(core_axis_name="c", subcore_axis_name="s")


@functools.partial(
    pl.kernel,
    mesh=_mesh,
    out_type=jax.ShapeDtypeStruct((SEQ, 2 * DIM), jnp.float32),
    scratch_types=[
        pltpu.VMEM((BPW,), jnp.int32),
        pltpu.VMEM((BPW, 2 * DIM), jnp.float32),
        pltpu.SemaphoreType.DMA,
        pltpu.SemaphoreType.DMA,
        pltpu.SemaphoreType.DMA,
    ],
)
def _rope_gather(tab_hbm, idx_hbm, wide_out, idx_v, rows_v, g0, g1, wsem):
    wid = lax.axis_index("s") * NC + lax.axis_index("c")
    base = wid * BPW
    pltpu.sync_copy(idx_hbm.at[0, pl.ds(base, BPW)], idx_v)
    gsems = (g0, g1)
    gathers = []
    for j in range(NCH):
        idx_sl = idx_v.at[pl.ds(j * CHUNK, CHUNK)]
        gathers.append(pltpu.async_copy(
            tab_hbm.at[idx_sl], rows_v.at[pl.ds(j * CHUNK, CHUNK)], gsems[j]))
    writes = []
    for j in range(NCH):
        gathers[j].wait()
        writes.append(pltpu.async_copy(
            rows_v.at[pl.ds(j * CHUNK, CHUNK)],
            wide_out.at[pl.ds(base + j * CHUNK, CHUNK)], wsem))
    for w in writes:
        w.wait()


_TR_ROWS = 1024


def _split_t_body(wide_ref, cos_ref, sin_ref):
    w = wide_ref[...]
    cos_ref[...] = w[:, :DIM].T
    sin_ref[...] = w[:, DIM:].T


_split_t = pl.pallas_call(
    _split_t_body,
    grid=(SEQ // _TR_ROWS,),
    in_specs=[pl.BlockSpec((_TR_ROWS, 2 * DIM), lambda i: (i, 0))],
    out_specs=[
        pl.BlockSpec((DIM, _TR_ROWS), lambda i: (0, i)),
        pl.BlockSpec((DIM, _TR_ROWS), lambda i: (0, i)),
    ],
    out_shape=[
        jax.ShapeDtypeStruct((DIM, SEQ), jnp.float32),
        jax.ShapeDtypeStruct((DIM, SEQ), jnp.float32),
    ],
)


def kernel(x, position_ids):
    tab = jnp.asarray(_TAB)
    idx = position_ids.astype(jnp.int32)
    wide = _rope_gather(tab, idx)
    cos_t, sin_t = _split_t(wide)
    cos = cos_t.T.reshape(1, 1, SEQ, DIM).astype(x.dtype)
    sin = sin_t.T.reshape(1, 1, SEQ, DIM).astype(x.dtype)
    return (cos, sin)


# speedup vs baseline: 1.2332x; 1.0187x over previous
import functools

import jax
import jax.numpy as jnp
import numpy as np
from jax import lax
from jax.experimental import pallas as pl
from jax.experimental.pallas import tpu as pltpu
from jax.experimental.pallas import tpu_sc as plsc

DIM = 64
MAX_POS = 8192
THETA = 10000.0
SEQ = 8192

NC = 2
NS = 16
NW = NC * NS
BPW = SEQ // NW
CHUNK = 128
NCH = BPW // CHUNK


def _fused_table():
    inv_freq = 1.0 / (THETA ** (np.arange(0, DIM, 2, dtype=np.float32) / DIM))
    t = np.arange(MAX_POS, dtype=np.float32)
    freqs = (t[:, None] * inv_freq[None, :]).astype(np.float32)
    emb = np.concatenate((freqs, freqs), axis=-1)
    tab = np.concatenate(
        (np.cos(emb), np.sin(emb)), axis=-1).astype(np.float32)
    return np.concatenate(
        (tab, np.zeros((8, 2 * DIM), np.float32)), axis=0)


_TAB = _fused_table()
_TAB_REF = jax.new_ref(jnp.asarray(_TAB))

_mesh = plsc.VectorSubcoreMesh(core_axis_name="c", subcore_axis_name="s")


@functools.partial(
    pl.kernel,
    mesh=_mesh,
    out_type=jax.ShapeDtypeStruct((SEQ, 2 * DIM), jnp.float32),
    scratch_types=[
        pltpu.VMEM((BPW,), jnp.int32),
        pltpu.VMEM((BPW, 2 * DIM), jnp.float32),
        pltpu.SemaphoreType.DMA,
        pltpu.SemaphoreType.DMA,
        pltpu.SemaphoreType.DMA,
    ],
)
def _rope_gather(tab_hbm, idx_hbm, wide_out, idx_v, rows_v, g0, g1, wsem):
    wid = lax.axis_index("s") * NC + lax.axis_index("c")
    base = wid * BPW
    pltpu.sync_copy(idx_hbm.at[0, pl.ds(base, BPW)], idx_v)
    gsems = (g0, g1)
    gathers = []
    for j in range(NCH):
        idx_sl = idx_v.at[pl.ds(j * CHUNK, CHUNK)]
        gathers.append(pltpu.async_copy(
            tab_hbm.at[idx_sl], rows_v.at[pl.ds(j * CHUNK, CHUNK)], gsems[j]))
    writes = []
    for j in range(NCH):
        gathers[j].wait()
        writes.append(pltpu.async_copy(
            rows_v.at[pl.ds(j * CHUNK, CHUNK)],
            wide_out.at[pl.ds(base + j * CHUNK, CHUNK)], wsem))
    for w in writes:
        w.wait()


_TR_ROWS = 1024


def _split_t_body(wide_ref, cos_ref, sin_ref):
    w = wide_ref[...]
    cos_ref[...] = w[:, :DIM].T
    sin_ref[...] = w[:, DIM:].T


_split_t = pl.pallas_call(
    _split_t_body,
    grid=(SEQ // _TR_ROWS,),
    in_specs=[pl.BlockSpec((_TR_ROWS, 2 * DIM), lambda i: (i, 0))],
    out_specs=[
        pl.BlockSpec((DIM, _TR_ROWS), lambda i: (0, i)),
        pl.BlockSpec((DIM, _TR_ROWS), lambda i: (0, i)),
    ],
    out_shape=[
        jax.ShapeDtypeStruct((DIM, SEQ), jnp.float32),
        jax.ShapeDtypeStruct((DIM, SEQ), jnp.float32),
    ],
)


def kernel(x, position_ids):
    idx = position_ids.astype(jnp.int32)
    wide = _rope_gather(_TAB_REF, idx)
    cos_t, sin_t = _split_t(wide)
    cos = cos_t.T.reshape(1, 1, SEQ, DIM).astype(x.dtype)
    sin = sin_t.T.reshape(1, 1, SEQ, DIM).astype(x.dtype)
    return (cos, sin)


# speedup vs baseline: 1.3178x; 1.0686x over previous
import functools

import jax
import jax.numpy as jnp
import numpy as np
from jax import lax
from jax.experimental import pallas as pl
from jax.experimental.pallas import tpu as pltpu
from jax.experimental.pallas import tpu_sc as plsc

DIM = 64
MAX_POS = 8192
THETA = 10000.0
SEQ = 8192

NC = 2
NS = 16
NW = NC * NS
BPW = SEQ // NW
CHUNK = 128
NCH = BPW // CHUNK


def _fused_table():
    inv_freq = 1.0 / (THETA ** (np.arange(0, DIM, 2, dtype=np.float32) / DIM))
    t = np.arange(MAX_POS, dtype=np.float32)
    freqs = (t[:, None] * inv_freq[None, :]).astype(np.float32)
    emb = np.concatenate((freqs, freqs), axis=-1)
    tab = np.concatenate(
        (np.cos(emb), np.sin(emb)), axis=-1).astype(np.float32)
    return np.concatenate(
        (tab, np.zeros((8, 2 * DIM), np.float32)), axis=0)


_TAB = _fused_table()
_TAB_REF = jax.new_ref(jnp.asarray(_TAB))

_mesh = plsc.VectorSubcoreMesh(core_axis_name="c", subcore_axis_name="s")


@functools.partial(
    pl.kernel,
    mesh=_mesh,
    out_type=jax.ShapeDtypeStruct((SEQ, 2 * DIM), jnp.float32),
    scratch_types=[
        pltpu.VMEM((BPW,), jnp.int32),
        pltpu.VMEM((BPW, 2 * DIM), jnp.float32),
        pltpu.SemaphoreType.DMA,
        pltpu.SemaphoreType.DMA,
        pltpu.SemaphoreType.DMA,
    ],
)
def _rope_gather(tab_hbm, idx_hbm, wide_out, idx_v, rows_v, g0, g1, wsem):
    wid = lax.axis_index("s") * NC + lax.axis_index("c")
    base = wid * BPW
    pltpu.sync_copy(idx_hbm.at[0, pl.ds(base, BPW)], idx_v)
    gsems = (g0, g1)
    gathers = []
    for j in range(NCH):
        idx_sl = idx_v.at[pl.ds(j * CHUNK, CHUNK)]
        gathers.append(pltpu.async_copy(
            tab_hbm.at[idx_sl], rows_v.at[pl.ds(j * CHUNK, CHUNK)], gsems[j]))
    writes = []
    for j in range(NCH):
        gathers[j].wait()
        writes.append(pltpu.async_copy(
            rows_v.at[pl.ds(j * CHUNK, CHUNK)],
            wide_out.at[pl.ds(base + j * CHUNK, CHUNK)], wsem))
    for w in writes:
        w.wait()


_TR_ROWS = 2048


def _split_t_body(wide_ref, cos_ref, sin_ref):
    w = wide_ref[...]
    cos_ref[...] = w[:, :DIM].T
    sin_ref[...] = w[:, DIM:].T


_split_t = pl.pallas_call(
    _split_t_body,
    grid=(SEQ // _TR_ROWS,),
    in_specs=[pl.BlockSpec((_TR_ROWS, 2 * DIM), lambda i: (i, 0))],
    out_specs=[
        pl.BlockSpec((DIM, _TR_ROWS), lambda i: (0, i)),
        pl.BlockSpec((DIM, _TR_ROWS), lambda i: (0, i)),
    ],
    out_shape=[
        jax.ShapeDtypeStruct((DIM, SEQ), jnp.float32),
        jax.ShapeDtypeStruct((DIM, SEQ), jnp.float32),
    ],
)


def kernel(x, position_ids):
    idx = position_ids.astype(jnp.int32)
    wide = _rope_gather(_TAB_REF, idx)
    cos_t, sin_t = _split_t(wide)
    cos = cos_t.T.reshape(1, 1, SEQ, DIM).astype(x.dtype)
    sin = sin_t.T.reshape(1, 1, SEQ, DIM).astype(x.dtype)
    return (cos, sin)


# speedup vs baseline: 1.3404x; 1.0172x over previous
import functools

import jax
import jax.numpy as jnp
import numpy as np
from jax import lax
from jax.experimental import pallas as pl
from jax.experimental.pallas import tpu as pltpu
from jax.experimental.pallas import tpu_sc as plsc

DIM = 64
MAX_POS = 8192
THETA = 10000.0
SEQ = 8192

NC = 2
NS = 16
NW = NC * NS
BPW = SEQ // NW
CHUNK = 128
NCH = BPW // CHUNK


def _fused_table():
    inv_freq = 1.0 / (THETA ** (np.arange(0, DIM, 2, dtype=np.float32) / DIM))
    t = np.arange(MAX_POS, dtype=np.float32)
    freqs = (t[:, None] * inv_freq[None, :]).astype(np.float32)
    emb = np.concatenate((freqs, freqs), axis=-1)
    tab = np.concatenate(
        (np.cos(emb), np.sin(emb)), axis=-1).astype(np.float32)
    return np.concatenate(
        (tab, np.zeros((8, 2 * DIM), np.float32)), axis=0)


_TAB = _fused_table()
_TAB_REF = jax.new_ref(jnp.asarray(_TAB))

_mesh = plsc.VectorSubcoreMesh(core_axis_name="c", subcore_axis_name="s")


@functools.partial(
    pl.kernel,
    mesh=_mesh,
    out_type=jax.ShapeDtypeStruct((SEQ, 2 * DIM), jnp.float32),
    scratch_types=[
        pltpu.VMEM((BPW,), jnp.int32),
        pltpu.VMEM((BPW, 2 * DIM), jnp.float32),
        pltpu.SemaphoreType.DMA,
        pltpu.SemaphoreType.DMA,
        pltpu.SemaphoreType.DMA,
    ],
)
def _rope_gather(tab_hbm, idx_hbm, wide_out, idx_v, rows_v, g0, g1, wsem):
    wid = lax.axis_index("s") * NC + lax.axis_index("c")
    base = wid * BPW
    pltpu.sync_copy(idx_hbm.at[0, pl.ds(base, BPW)], idx_v)
    gsems = (g0, g1)
    gathers = []
    for j in range(NCH):
        idx_sl = idx_v.at[pl.ds(j * CHUNK, CHUNK)]
        gathers.append(pltpu.async_copy(
            tab_hbm.at[idx_sl], rows_v.at[pl.ds(j * CHUNK, CHUNK)], gsems[j]))
    writes = []
    for j in range(NCH):
        gathers[j].wait()
        writes.append(pltpu.async_copy(
            rows_v.at[pl.ds(j * CHUNK, CHUNK)],
            wide_out.at[pl.ds(base + j * CHUNK, CHUNK)], wsem))
    for w in writes:
        w.wait()


_TR_ROWS = 4096


def _split_t_body(wide_ref, cos_ref, sin_ref):
    w = wide_ref[...]
    cos_ref[...] = w[:, :DIM].T
    sin_ref[...] = w[:, DIM:].T


_split_t = pl.pallas_call(
    _split_t_body,
    grid=(SEQ // _TR_ROWS,),
    in_specs=[pl.BlockSpec((_TR_ROWS, 2 * DIM), lambda i: (i, 0))],
    out_specs=[
        pl.BlockSpec((DIM, _TR_ROWS), lambda i: (0, i)),
        pl.BlockSpec((DIM, _TR_ROWS), lambda i: (0, i)),
    ],
    out_shape=[
        jax.ShapeDtypeStruct((DIM, SEQ), jnp.float32),
        jax.ShapeDtypeStruct((DIM, SEQ), jnp.float32),
    ],
)


def kernel(x, position_ids):
    idx = position_ids.astype(jnp.int32)
    wide = _rope_gather(_TAB_REF, idx)
    cos_t, sin_t = _split_t(wide)
    cos = cos_t.T.reshape(1, 1, SEQ, DIM).astype(x.dtype)
    sin = sin_t.T.reshape(1, 1, SEQ, DIM).astype(x.dtype)
    return (cos, sin)
